# speedup vs baseline: 1.2270x; 1.2270x over previous
import jax
import jax.numpy as jnp
from jax.experimental import pallas as pl
from jax.experimental.pallas import tpu as pltpu

_TRUE_HIDDEN = 32
_SUBLANE = 8
_ROW_TILE = 8192


def _ceil_to(x: int, m: int) -> int:
    return ((x + m - 1) // m) * m


def _packed_mlp_body(xr_ref, w1bd_ref, b1t_ref, w2bd_ref, b2t_ref, out_ref):
    h = jax.lax.dot_general(
        xr_ref[...], w1bd_ref[...],
        (((1,), (0,)), ((), ())),
        preferred_element_type=jnp.float32,
    )
    h = jnp.maximum(h + b1t_ref[...], 0.0)
    y = jax.lax.dot_general(
        h, w2bd_ref[...],
        (((1,), (0,)), ((), ())),
        preferred_element_type=jnp.float32,
    )
    out_ref[...] = y + b2t_ref[...]


def kernel(x, w1p, b1p, w2p, b2p):
    B, in_w = x.shape
    hid_p = w1p.shape[1]
    out_w = w2p.shape[1]
    x = x.astype(jnp.float32)

    if B == 0:
        return jnp.zeros((0, out_w), jnp.float32)

    h_w = min(_TRUE_HIDDEN, hid_p)
    w1s = w1p[:, :h_w].astype(jnp.float32)
    b1s = b1p[:, :h_w].astype(jnp.float32)
    w2s = w2p[:h_w, :].astype(jnp.float32)
    b2s = b2p.reshape(1, out_w).astype(jnp.float32)

    r = _SUBLANE if B % _SUBLANE == 0 else 1
    packed_rows = B // r
    ident = jnp.eye(r, dtype=jnp.float32)
    w1bd = jnp.kron(ident, w1s)
    w2bd = jnp.kron(ident, w2s)
    b1t = jnp.tile(b1s, (1, r))
    b2t = jnp.tile(b2s, (1, r))
    xr = x.reshape(packed_rows, r * in_w)

    tile = min(_ROW_TILE, max(_SUBLANE, _ceil_to(pl.cdiv(packed_rows, 2), _SUBLANE)))
    grid = (pl.cdiv(packed_rows, tile),)

    packed_out = pl.pallas_call(
        _packed_mlp_body,
        out_shape=jax.ShapeDtypeStruct((packed_rows, r * out_w), jnp.float32),
        grid=grid,
        in_specs=[
            pl.BlockSpec((tile, r * in_w), lambda i: (i, 0)),
            pl.BlockSpec((r * in_w, r * h_w), lambda i: (0, 0)),
            pl.BlockSpec((1, r * h_w), lambda i: (0, 0)),
            pl.BlockSpec((r * h_w, r * out_w), lambda i: (0, 0)),
            pl.BlockSpec((1, r * out_w), lambda i: (0, 0)),
        ],
        out_specs=pl.BlockSpec((tile, r * out_w), lambda i: (i, 0)),
        compiler_params=pltpu.CompilerParams(
            dimension_semantics=("parallel",),
        ),
    )(xr, w1bd, b1t, w2bd, b2t)

    return packed_out.reshape(B, out_w)


# speedup vs baseline: 1.3113x; 1.0687x over previous
import jax
import jax.numpy as jnp
from jax.experimental import pallas as pl
from jax.experimental.pallas import tpu as pltpu

_TRUE_HIDDEN = 32
_SUBLANE = 8
_BATCH_TILE = 16384


def _ceil_to(x: int, m: int) -> int:
    return ((x + m - 1) // m) * m


def _qnet_body(x_ref, w1_ref, b1_ref, w2_ref, b2_ref, o_ref):
    xb = x_ref[...].astype(jnp.bfloat16)
    pre = jax.lax.dot_general(
        xb, w1_ref[...],
        (((1,), (0,)), ((), ())),
        preferred_element_type=jnp.float32,
    )
    h = jnp.maximum(pre + b1_ref[...], 0.0).astype(jnp.bfloat16)
    y = jax.lax.dot_general(
        h, w2_ref[...],
        (((1,), (0,)), ((), ())),
        preferred_element_type=jnp.float32,
    )
    o_ref[...] = y + b2_ref[...]


def kernel(x, w1p, b1p, w2p, b2p):
    B, in_w = x.shape
    hid_p = w1p.shape[1]
    out_w = w2p.shape[1]
    x = x.astype(jnp.float32)

    if B == 0:
        return jnp.zeros((0, out_w), jnp.float32)

    h_w = min(_TRUE_HIDDEN, hid_p)
    w1s = w1p[:, :h_w].astype(jnp.bfloat16)
    b1s = b1p[:, :h_w].astype(jnp.float32)
    w2s = w2p[:h_w, :].astype(jnp.bfloat16)
    b2s = b2p.reshape(1, out_w).astype(jnp.float32)

    tile = min(_BATCH_TILE, max(_SUBLANE, _ceil_to(pl.cdiv(B, 2), _SUBLANE)))
    grid = (pl.cdiv(B, tile),)

    return pl.pallas_call(
        _qnet_body,
        out_shape=jax.ShapeDtypeStruct((B, out_w), jnp.float32),
        grid=grid,
        in_specs=[
            pl.BlockSpec((tile, in_w), lambda i: (i, 0)),
            pl.BlockSpec((in_w, h_w), lambda i: (0, 0)),
            pl.BlockSpec((1, h_w), lambda i: (0, 0)),
            pl.BlockSpec((h_w, out_w), lambda i: (0, 0)),
            pl.BlockSpec((1, out_w), lambda i: (0, 0)),
        ],
        out_specs=pl.BlockSpec((tile, out_w), lambda i: (i, 0)),
        compiler_params=pltpu.CompilerParams(
            dimension_semantics=("parallel",),
        ),
    )(x, w1s, b1s, w2s, b2s)


# speedup vs baseline: 28.2597x; 21.5507x over previous
import jax
import jax.numpy as jnp
from jax.experimental import pallas as pl
from jax.experimental.pallas import tpu as pltpu

_TRUE_HIDDEN = 32
_LANE = 128
_COL_TILE = 65536


def _ceil_to(x: int, m: int) -> int:
    return ((x + m - 1) // m) * m


def _tmlp_body(xT_ref, w1T_ref, b1T_ref, w2T_ref, b2T_ref, oT_ref):
    xb = xT_ref[...].astype(jnp.bfloat16)
    h = jax.lax.dot_general(
        w1T_ref[...], xb,
        (((1,), (0,)), ((), ())),
        preferred_element_type=jnp.float32,
    )
    h = jnp.maximum(h + b1T_ref[...], 0.0).astype(jnp.bfloat16)
    y = jax.lax.dot_general(
        w2T_ref[...], h,
        (((1,), (0,)), ((), ())),
        preferred_element_type=jnp.float32,
    )
    oT_ref[...] = y + b2T_ref[...]


def kernel(x, w1p, b1p, w2p, b2p):
    B, in_w = x.shape
    hid_p = w1p.shape[1]
    out_w = w2p.shape[1]
    x = x.astype(jnp.float32)

    if B == 0:
        return jnp.zeros((0, out_w), jnp.float32)

    h_w = min(_TRUE_HIDDEN, hid_p)
    w1T = w1p[:, :h_w].T.astype(jnp.bfloat16)
    b1T = b1p[:, :h_w].T.astype(jnp.float32)
    w2T = w2p[:h_w, :].T.astype(jnp.bfloat16)
    b2T = b2p.reshape(1, out_w).T.astype(jnp.float32)

    xT = x.T

    tile = min(_COL_TILE, max(_LANE, _ceil_to(pl.cdiv(B, 2), _LANE)))
    grid = (pl.cdiv(B, tile),)

    yT = pl.pallas_call(
        _tmlp_body,
        out_shape=jax.ShapeDtypeStruct((out_w, B), jnp.float32),
        grid=grid,
        in_specs=[
            pl.BlockSpec((in_w, tile), lambda i: (0, i)),
            pl.BlockSpec((h_w, in_w), lambda i: (0, 0)),
            pl.BlockSpec((h_w, 1), lambda i: (0, 0)),
            pl.BlockSpec((out_w, h_w), lambda i: (0, 0)),
            pl.BlockSpec((out_w, 1), lambda i: (0, 0)),
        ],
        out_specs=pl.BlockSpec((out_w, tile), lambda i: (0, i)),
        compiler_params=pltpu.CompilerParams(
            dimension_semantics=("parallel",),
        ),
    )(xT, w1T, b1T, w2T, b2T)

    return yT.T
